# baseline (speedup 1.0000x reference)
import functools
import math

import jax
import jax.numpy as jnp
from jax import lax
from jax.experimental import pallas as pl
from jax.experimental.pallas import tpu as pltpu
from jax.experimental.pallas import tpu_sc as plsc

VOCAB = 1000000
EMBED_DIM = 64
SCALE = math.sqrt(EMBED_DIM)

_INFO = plsc.get_sparse_core_info()
NC = _INFO.num_cores
NS = _INFO.num_subcores
NW = NC * NS
LANES = _INFO.num_lanes

B_TOTAL = 4096 * 200
B_PER_W = B_TOTAL // NW
CHUNK = 512
N_CHUNKS = B_PER_W // CHUNK
SLICES_PER_ROW = EMBED_DIM // LANES


def _make_gather_kernel():
    mesh = plsc.VectorSubcoreMesh(core_axis_name="c", subcore_axis_name="s")

    @functools.partial(
        pl.kernel,
        out_type=jax.ShapeDtypeStruct((B_TOTAL, EMBED_DIM), jnp.float32),
        mesh=mesh,
        scratch_types=[
            pltpu.VMEM((CHUNK,), jnp.int32),
            pltpu.VMEM((CHUNK, EMBED_DIM), jnp.float32),
            pltpu.SemaphoreType.DMA,
        ],
        compiler_params=pltpu.CompilerParams(use_tc_tiling_on_sc=False),
    )
    def gather_kernel(x_hbm, table_hbm, out_hbm, idx_v, rows_v, sem):
        wid = lax.axis_index("s") * NC + lax.axis_index("c")
        base = wid * B_PER_W

        def chunk_body(i, carry):
            off = base + i * CHUNK
            pltpu.sync_copy(x_hbm.at[pl.ds(off, CHUNK)], idx_v)
            pltpu.async_copy(table_hbm.at[idx_v], rows_v, sem).wait()

            def scale_row(r, c2):
                for j in range(SLICES_PER_ROW):
                    sl = (r, pl.ds(j * LANES, LANES))
                    rows_v[sl] = rows_v[sl] * SCALE
                return c2

            lax.fori_loop(0, CHUNK, scale_row, 0, unroll=2)
            pltpu.sync_copy(rows_v, out_hbm.at[pl.ds(off, CHUNK)])
            return carry

        lax.fori_loop(0, N_CHUNKS, chunk_body, 0)

    return gather_kernel


_GATHER = _make_gather_kernel()


@jax.jit
def kernel(x, table):
    flat_idx = x.reshape(-1)
    out = _GATHER(flat_idx, table)
    return out.reshape(x.shape[0], x.shape[1], EMBED_DIM)


# speedup vs baseline: 1.0931x; 1.0931x over previous
import functools
import math

import jax
import jax.numpy as jnp
from jax import lax
from jax.experimental import pallas as pl
from jax.experimental.pallas import tpu as pltpu
from jax.experimental.pallas import tpu_sc as plsc

VOCAB = 1000000
EMBED_DIM = 64
SCALE = math.sqrt(EMBED_DIM)

_INFO = plsc.get_sparse_core_info()
NC = _INFO.num_cores
NS = _INFO.num_subcores
NW = NC * NS
LANES = _INFO.num_lanes

B_TOTAL = 4096 * 200
B_PER_W = B_TOTAL // NW
CHUNK = 256
N_CHUNKS = B_PER_W // CHUNK
NBUF = 4
LOOKAHEAD = 2
SLICES_PER_ROW = EMBED_DIM // LANES


def _make_gather_kernel():
    mesh = plsc.VectorSubcoreMesh(core_axis_name="c", subcore_axis_name="s")

    @functools.partial(
        pl.kernel,
        out_type=jax.ShapeDtypeStruct((B_TOTAL, EMBED_DIM), jnp.float32),
        mesh=mesh,
        scratch_types=[
            pltpu.VMEM((B_PER_W,), jnp.int32),
            [pltpu.VMEM((CHUNK, EMBED_DIM), jnp.float32) for _ in range(NBUF)],
            [pltpu.SemaphoreType.DMA for _ in range(NBUF)],
            [pltpu.SemaphoreType.DMA for _ in range(NBUF)],
            pltpu.SemaphoreType.DMA,
        ],
        compiler_params=pltpu.CompilerParams(use_tc_tiling_on_sc=False),
    )
    def gather_kernel(x_hbm, table_hbm, out_hbm, idx_v, rows, sem_g, sem_s,
                      sem_i):
        wid = lax.axis_index("s") * NC + lax.axis_index("c")
        base = wid * B_PER_W

        pltpu.async_copy(x_hbm.at[pl.ds(base, B_PER_W)], idx_v, sem_i).wait()

        def idx_slice(c):
            return idx_v.at[pl.ds(c * CHUNK, CHUNK)]

        def out_slice(c):
            return out_hbm.at[pl.ds(base + c * CHUNK, CHUNK)]

        def start_gather(c, b):
            pltpu.async_copy(table_hbm.at[idx_slice(c)], rows[b], sem_g[b])

        def wait_gather(c, b):
            pltpu.make_async_copy(table_hbm.at[idx_slice(c)], rows[b],
                                  sem_g[b]).wait()

        def start_scatter(c, b):
            pltpu.async_copy(rows[b], out_slice(c), sem_s[b])

        def wait_scatter(c, b):
            pltpu.make_async_copy(rows[b], out_slice(c), sem_s[b]).wait()

        for c in range(LOOKAHEAD):
            start_gather(c, c % NBUF)

        @pl.loop(0, N_CHUNKS, step=NBUF)
        def chunk_group(i):
            for db in range(NBUF):
                c = i + db
                b = db
                bn = (db + LOOKAHEAD) % NBUF

                @pl.when(c + LOOKAHEAD < N_CHUNKS)
                def _():
                    @pl.when(c >= NBUF - LOOKAHEAD)
                    def _():
                        wait_scatter(c - (NBUF - LOOKAHEAD), bn)
                    start_gather(c + LOOKAHEAD, bn)

                wait_gather(c, b)

                @plsc.parallel_loop(0, CHUNK, unroll=4)
                def scale_row(r):
                    for j in range(SLICES_PER_ROW):
                        sl = (r, pl.ds(j * LANES, LANES))
                        rows[b][sl] = rows[b][sl] * SCALE

                start_scatter(c, b)

        for k in range(NBUF):
            c = N_CHUNKS - NBUF + k
            wait_scatter(c, c % NBUF)

    return gather_kernel


_GATHER = _make_gather_kernel()


@jax.jit
def kernel(x, table):
    flat_idx = x.reshape(-1)
    out = _GATHER(flat_idx, table)
    return out.reshape(x.shape[0], x.shape[1], EMBED_DIM)
